# baseline (speedup 1.0000x reference)
import functools

import jax
import jax.numpy as jnp
from jax import lax
from jax.experimental import pallas as pl
from jax.experimental.pallas import tpu as pltpu
from jax.experimental.pallas import tpu_sc as plsc

_N = 10000
_D = 128
_BR = 1000
_BC = 2048
_NCB = 5

_WK = 25
_RPW = 400
_CH = 200

_BR1 = 2000
_BR2 = 2000


def _top2_body(a_ref, out_ref, t1s, i1s, t2s, i2s):
    j = pl.program_id(1)
    nj = pl.num_programs(1)
    lanef = lax.broadcasted_iota(
        jnp.int32, (_BR, 128), 1).astype(jnp.float32)

    @pl.when(j == 0)
    def _init():
        t1s[...] = jnp.full((_BR, 128), -jnp.inf, jnp.float32)
        t2s[...] = jnp.full((_BR, 128), -jnp.inf, jnp.float32)
        i1s[...] = jnp.zeros((_BR, 128), jnp.float32)
        i2s[...] = jnp.zeros((_BR, 128), jnp.float32)

    t1, i1 = t1s[...], i1s[...]
    t2, i2 = t2s[...], i2s[...]
    for s in range(_BC // 128):
        v = a_ref[:, pl.ds(s * 128, 128)]
        lim = (_N - j * _BC - s * 128).astype(jnp.float32)
        v = jnp.where(lanef < lim, v, -jnp.inf)
        bid = (j * (_BC // 128) + s).astype(jnp.float32)
        gt1 = v > t1
        cv = jnp.where(gt1, t1, v)
        ci = jnp.where(gt1, i1, bid)
        t1 = jnp.where(gt1, v, t1)
        i1 = jnp.where(gt1, bid, i1)
        gt2 = cv > t2
        t2 = jnp.where(gt2, cv, t2)
        i2 = jnp.where(gt2, ci, i2)
    t1s[...], i1s[...] = t1, i1
    t2s[...], i2s[...] = t2, i2

    @pl.when(j == nj - 1)
    def _write():
        bigf = jnp.float32(1e9)
        c1 = i1 * 128.0 + lanef
        c2 = i2 * 128.0 + lanef
        m1 = jnp.max(t1, axis=1, keepdims=True)
        i1row = jnp.min(jnp.where(t1 == m1, c1, bigf), axis=1, keepdims=True)
        w = (t1 == m1) & (c1 == i1row)
        t1x = jnp.where(w, t2, t1)
        c1x = jnp.where(w, c2, c1)
        m2 = jnp.max(t1x, axis=1, keepdims=True)
        i2row = jnp.min(jnp.where(t1x == m2, c1x, bigf), axis=1,
                        keepdims=True)
        lane8 = lax.broadcasted_iota(jnp.int32, (out_ref.shape[0], 8), 1)
        out_ref[...] = jnp.where(lane8 == 0, i1row.astype(jnp.int32),
                                 jnp.where(lane8 == 1,
                                           i2row.astype(jnp.int32), 0))


def _top2(adjacency):
    return pl.pallas_call(
        _top2_body,
        grid=(_N // _BR, _NCB),
        in_specs=[pl.BlockSpec((_BR, _BC), lambda i, j: (i, j))],
        out_specs=pl.BlockSpec((_BR, 8), lambda i, j: (i, 0)),
        out_shape=jax.ShapeDtypeStruct((_N, 8), jnp.int32),
        scratch_shapes=[
            pltpu.VMEM((_BR, 128), jnp.float32),
            pltpu.VMEM((_BR, 128), jnp.float32),
            pltpu.VMEM((_BR, 128), jnp.float32),
            pltpu.VMEM((_BR, 128), jnp.float32),
        ],
        compiler_params=pltpu.CompilerParams(
            dimension_semantics=("parallel", "arbitrary")),
    )(adjacency)


def _sc_combine_body(table_hbm, i0_hbm, i1_hbm, out_hbm,
                     i0_v, i1_v, self_v, g0_v, g1_v, sem0, sem1, sem2):
    wid = lax.axis_index("s") * 2 + lax.axis_index("c")

    @pl.when(wid < _WK)
    def _work():
        for s in range(_RPW // _CH):
            b = wid * _RPW + s * _CH
            cp0 = pltpu.async_copy(i0_hbm.at[pl.ds(b, _CH)], i0_v, sem0)
            cp1 = pltpu.async_copy(i1_hbm.at[pl.ds(b, _CH)], i1_v, sem1)
            cps = pltpu.async_copy(table_hbm.at[pl.ds(b, _CH)], self_v, sem2)
            cp0.wait()
            cp1.wait()
            g0c = pltpu.async_copy(table_hbm.at[i0_v], g0_v, sem0)
            g1c = pltpu.async_copy(table_hbm.at[i1_v], g1_v, sem1)
            cps.wait()
            g0c.wait()
            g1c.wait()

            def row(r, _):
                for c in range(_D // 16):
                    sl = pl.ds(c * 16, 16)
                    agg = (g0_v[r, sl] + g1_v[r, sl]) * 0.5
                    self_v[r, sl] = (self_v[r, sl] + agg) * 0.5
                return 0

            lax.fori_loop(0, _CH, row, 0)
            pltpu.sync_copy(self_v, out_hbm.at[pl.ds(b, _CH)])


def _sc_combine(table, i0, i1):
    mesh = plsc.VectorSubcoreMesh(core_axis_name="c", subcore_axis_name="s")
    fn = functools.partial(
        pl.kernel,
        out_type=jax.ShapeDtypeStruct((_N, _D), jnp.float32),
        mesh=mesh,
        scratch_types=[
            pltpu.VMEM((_CH,), jnp.int32),
            pltpu.VMEM((_CH,), jnp.int32),
            pltpu.VMEM((_CH, _D), jnp.float32),
            pltpu.VMEM((_CH, _D), jnp.float32),
            pltpu.VMEM((_CH, _D), jnp.float32),
            pltpu.SemaphoreType.DMA,
            pltpu.SemaphoreType.DMA,
            pltpu.SemaphoreType.DMA,
        ],
    )(_sc_combine_body)
    return fn(table, i0, i1)


def _ens_logits(f, wih_ref, bh_ref, who_ref, bo_ref, m):
    hid = jnp.maximum(
        jnp.dot(f, wih_ref[m], preferred_element_type=jnp.float32)
        + bh_ref[m][None, :], 0.0)
    return (jnp.dot(hid, who_ref[m], preferred_element_type=jnp.float32)
            + bo_ref[m][None, :])


def _mlp1_body(f_ref, wih_ref, bh_ref, who_ref, bo_ref, mx_ref):
    f = f_ref[...]
    bms = []
    for m in range(4):
        lg = _ens_logits(f, wih_ref, bh_ref, who_ref, bo_ref, m)
        bms.append(jnp.max(lg, axis=0))
    mx_ref[0] = jnp.stack(bms, axis=0)


def _mlp1(f2, wih, bh, who, bo):
    c = who.shape[2]
    nb = _N // _BR1
    return pl.pallas_call(
        _mlp1_body,
        grid=(nb,),
        in_specs=[
            pl.BlockSpec((_BR1, _D), lambda i: (i, 0)),
            pl.BlockSpec(wih.shape, lambda i: (0, 0, 0)),
            pl.BlockSpec(bh.shape, lambda i: (0, 0)),
            pl.BlockSpec(who.shape, lambda i: (0, 0, 0)),
            pl.BlockSpec(bo.shape, lambda i: (0, 0)),
        ],
        out_specs=pl.BlockSpec((1, 4, c), lambda i: (i, 0, 0)),
        out_shape=jax.ShapeDtypeStruct((nb, 4, c), jnp.float32),
        compiler_params=pltpu.CompilerParams(
            dimension_semantics=("parallel",)),
    )(f2, wih, bh, who, bo)


def _mlp2a_body(mx_ref, f_ref, ewih_ref, ebh_ref, ewho_ref, ebo_ref,
                wih_ref, bh_ref, who_ref, bo_ref, o_ref, gm_ref):
    c = mx_ref.shape[2]
    mxv = jnp.max(mx_ref[...], axis=0)
    f = f_ref[...]
    preds_sum = jnp.zeros((f_ref.shape[0], 1), jnp.float32)
    for m in range(4):
        lg = _ens_logits(f, ewih_ref, ebh_ref, ewho_ref, ebo_ref, m)
        e = jnp.exp(lg - jnp.max(mxv[m]))
        p = e / jnp.sum(e, axis=1, keepdims=True)
        pm = jnp.max(p, axis=1, keepdims=True)
        colc = lax.broadcasted_iota(jnp.int32, p.shape, 1)
        am = jnp.min(jnp.where(p == pm, colc, jnp.int32(c)), axis=1,
                     keepdims=True)
        am = jnp.where(jnp.isnan(pm), jnp.int32(0), am)
        preds_sum = preds_sum + am.astype(jnp.float32)
    agg = preds_sum * 0.25
    w = f * agg
    h2 = jnp.maximum(
        jnp.dot(w, wih_ref[...], preferred_element_type=jnp.float32)
        + bh_ref[...][None, :], 0.0)
    o = (jnp.dot(h2, who_ref[...], preferred_element_type=jnp.float32)
         + bo_ref[...][None, :])
    o_ref[...] = o
    gm_ref[...] = jnp.broadcast_to(jnp.max(o), gm_ref.shape)


def _mlp2a(mx, f2, ewih, ebh, ewho, ebo, wih, bh, who, bo):
    c = mx.shape[2]
    nb = _N // _BR2
    return pl.pallas_call(
        _mlp2a_body,
        grid=(nb,),
        in_specs=[
            pl.BlockSpec(mx.shape, lambda i: (0, 0, 0)),
            pl.BlockSpec((_BR2, _D), lambda i: (i, 0)),
            pl.BlockSpec(ewih.shape, lambda i: (0, 0, 0)),
            pl.BlockSpec(ebh.shape, lambda i: (0, 0)),
            pl.BlockSpec(ewho.shape, lambda i: (0, 0, 0)),
            pl.BlockSpec(ebo.shape, lambda i: (0, 0)),
            pl.BlockSpec(wih.shape, lambda i: (0, 0)),
            pl.BlockSpec(bh.shape, lambda i: (0,)),
            pl.BlockSpec(who.shape, lambda i: (0, 0)),
            pl.BlockSpec(bo.shape, lambda i: (0,)),
        ],
        out_specs=[
            pl.BlockSpec((_BR2, c), lambda i: (i, 0)),
            pl.BlockSpec((1, 8, 128), lambda i: (i, 0, 0)),
        ],
        out_shape=[
            jax.ShapeDtypeStruct((_N, c), jnp.float32),
            jax.ShapeDtypeStruct((nb, 8, 128), jnp.float32),
        ],
        compiler_params=pltpu.CompilerParams(
            dimension_semantics=("parallel",)),
    )(mx, f2, ewih, ebh, ewho, ebo, wih, bh, who, bo)


def _mlp2b_body(o_ref, gm_ref, out_ref):
    g = jnp.max(gm_ref[...])
    e2 = jnp.exp(o_ref[...] - g)
    out_ref[...] = e2 / jnp.sum(e2, axis=1, keepdims=True)


def _mlp2b(o, gm):
    c = o.shape[1]
    return pl.pallas_call(
        _mlp2b_body,
        grid=(_N // _BR2,),
        in_specs=[
            pl.BlockSpec((_BR2, c), lambda i: (i, 0)),
            pl.BlockSpec(gm.shape, lambda i: (0, 0, 0)),
        ],
        out_specs=pl.BlockSpec((_BR2, c), lambda i: (i, 0)),
        out_shape=jax.ShapeDtypeStruct((_N, c), jnp.float32),
        compiler_params=pltpu.CompilerParams(
            dimension_semantics=("parallel",)),
    )(o, gm)


def kernel(adjacency_matrix, node_features, ens_W_ih, ens_b_h, ens_W_ho,
           ens_b_o, clf_W_ih, clf_b_h, clf_W_ho, clf_b_o):
    idx8 = _top2(adjacency_matrix)
    i0 = idx8[:, 0]
    i1 = idx8[:, 1]
    f1 = _sc_combine(node_features, i0, i1)
    f2 = _sc_combine(f1, i0, i1)
    mx = _mlp1(f2, ens_W_ih, ens_b_h, ens_W_ho, ens_b_o)
    o, gm = _mlp2a(mx, f2, ens_W_ih, ens_b_h, ens_W_ho, ens_b_o,
                   clf_W_ih, clf_b_h, clf_W_ho, clf_b_o)
    return _mlp2b(o, gm)


# speedup vs baseline: 1.1376x; 1.1376x over previous
import functools

import jax
import jax.numpy as jnp
from jax import lax
from jax.experimental import pallas as pl
from jax.experimental.pallas import tpu as pltpu
from jax.experimental.pallas import tpu_sc as plsc

_N = 10000
_D = 128
_BR = 1000
_BC = 2048
_NCB = 5

_WK = 25
_RPW = 400
_CH = 200

_BR1 = 1000
_BR2 = 1000


def _top2_body(a_ref, out_ref, rv1, ri1, rv2, ri2):
    j = pl.program_id(1)
    nj = pl.num_programs(1)
    v = a_ref[...]
    colf = lax.broadcasted_iota(jnp.int32, v.shape, 1).astype(jnp.float32)
    limit = (_N - j * _BC).astype(jnp.float32)
    v = jnp.where(colf < limit, v, -jnp.inf)
    bigf = jnp.float32(4096.0)
    m1 = jnp.max(v, axis=1, keepdims=True)
    i1f = jnp.min(jnp.where(v == m1, colf, bigf), axis=1, keepdims=True)
    v2 = jnp.where(colf == i1f, -jnp.inf, v)
    m2 = jnp.max(v2, axis=1, keepdims=True)
    i2f = jnp.min(jnp.where(v2 == m2, colf, bigf), axis=1, keepdims=True)
    off = j * _BC
    i1 = i1f.astype(jnp.int32) + off
    i2 = i2f.astype(jnp.int32) + off

    @pl.when(j == 0)
    def _init():
        rv1[...] = m1
        ri1[...] = i1
        rv2[...] = m2
        ri2[...] = i2

    @pl.when(j > 0)
    def _merge():
        pv1, pi1 = rv1[...], ri1[...]
        pv2, pi2 = rv2[...], ri2[...]
        b1 = m1 > pv1
        cav = jnp.where(b1, pv1, pv2)
        cai = jnp.where(b1, pi1, pi2)
        cbv = jnp.where(b1, m2, m1)
        cbi = jnp.where(b1, i2, i1)
        b2 = cbv > cav
        rv1[...] = jnp.where(b1, m1, pv1)
        ri1[...] = jnp.where(b1, i1, pi1)
        rv2[...] = jnp.where(b2, cbv, cav)
        ri2[...] = jnp.where(b2, cbi, cai)

    @pl.when(j == nj - 1)
    def _write():
        lane = lax.broadcasted_iota(jnp.int32, (out_ref.shape[0], 8), 1)
        out_ref[...] = jnp.where(lane == 0, ri1[...],
                                 jnp.where(lane == 1, ri2[...], 0))


def _top2(adjacency):
    return pl.pallas_call(
        _top2_body,
        grid=(_N // _BR, _NCB),
        in_specs=[pl.BlockSpec((_BR, _BC), lambda i, j: (i, j))],
        out_specs=pl.BlockSpec((_BR, 8), lambda i, j: (i, 0)),
        out_shape=jax.ShapeDtypeStruct((_N, 8), jnp.int32),
        scratch_shapes=[
            pltpu.VMEM((_BR, 1), jnp.float32),
            pltpu.VMEM((_BR, 1), jnp.int32),
            pltpu.VMEM((_BR, 1), jnp.float32),
            pltpu.VMEM((_BR, 1), jnp.int32),
        ],
        compiler_params=pltpu.CompilerParams(
            dimension_semantics=("parallel", "arbitrary")),
    )(adjacency)


def _sc_combine_body(table_hbm, i0_hbm, i1_hbm, out_hbm,
                     i0_v, i1_v, self_v, g0_v, g1_v, sem0, sem1, sem2):
    wid = lax.axis_index("s") * 2 + lax.axis_index("c")

    @pl.when(wid < _WK)
    def _work():
        for s in range(_RPW // _CH):
            b = wid * _RPW + s * _CH
            cp0 = pltpu.async_copy(i0_hbm.at[pl.ds(b, _CH)], i0_v, sem0)
            cp1 = pltpu.async_copy(i1_hbm.at[pl.ds(b, _CH)], i1_v, sem1)
            cps = pltpu.async_copy(table_hbm.at[pl.ds(b, _CH)], self_v, sem2)
            cp0.wait()
            cp1.wait()
            g0c = pltpu.async_copy(table_hbm.at[i0_v], g0_v, sem0)
            g1c = pltpu.async_copy(table_hbm.at[i1_v], g1_v, sem1)
            cps.wait()
            g0c.wait()
            g1c.wait()

            def row(r, _):
                for c in range(_D // 16):
                    sl = pl.ds(c * 16, 16)
                    agg = (g0_v[r, sl] + g1_v[r, sl]) * 0.5
                    self_v[r, sl] = (self_v[r, sl] + agg) * 0.5
                return 0

            lax.fori_loop(0, _CH, row, 0)
            pltpu.sync_copy(self_v, out_hbm.at[pl.ds(b, _CH)])


def _sc_combine(table, i0, i1):
    mesh = plsc.VectorSubcoreMesh(core_axis_name="c", subcore_axis_name="s")
    fn = functools.partial(
        pl.kernel,
        out_type=jax.ShapeDtypeStruct((_N, _D), jnp.float32),
        mesh=mesh,
        scratch_types=[
            pltpu.VMEM((_CH,), jnp.int32),
            pltpu.VMEM((_CH,), jnp.int32),
            pltpu.VMEM((_CH, _D), jnp.float32),
            pltpu.VMEM((_CH, _D), jnp.float32),
            pltpu.VMEM((_CH, _D), jnp.float32),
            pltpu.SemaphoreType.DMA,
            pltpu.SemaphoreType.DMA,
            pltpu.SemaphoreType.DMA,
        ],
    )(_sc_combine_body)
    return fn(table, i0, i1)


def _ens_logits(f, wih_ref, bh_ref, who_ref, bo_ref, m):
    hid = jnp.maximum(
        jnp.dot(f, wih_ref[m], preferred_element_type=jnp.float32)
        + bh_ref[m][None, :], 0.0)
    return (jnp.dot(hid, who_ref[m], preferred_element_type=jnp.float32)
            + bo_ref[m][None, :])


def _mlp1_body(f_ref, wih_ref, bh_ref, who_ref, bo_ref, mx_ref):
    f = f_ref[...]
    bms = []
    for m in range(4):
        lg = _ens_logits(f, wih_ref, bh_ref, who_ref, bo_ref, m)
        bms.append(jnp.max(lg, axis=0))
    mx_ref[0] = jnp.stack(bms, axis=0)


def _mlp1(f2, wih, bh, who, bo):
    c = who.shape[2]
    nb = _N // _BR1
    return pl.pallas_call(
        _mlp1_body,
        grid=(nb,),
        in_specs=[
            pl.BlockSpec((_BR1, _D), lambda i: (i, 0)),
            pl.BlockSpec(wih.shape, lambda i: (0, 0, 0)),
            pl.BlockSpec(bh.shape, lambda i: (0, 0)),
            pl.BlockSpec(who.shape, lambda i: (0, 0, 0)),
            pl.BlockSpec(bo.shape, lambda i: (0, 0)),
        ],
        out_specs=pl.BlockSpec((1, 4, c), lambda i: (i, 0, 0)),
        out_shape=jax.ShapeDtypeStruct((nb, 4, c), jnp.float32),
        compiler_params=pltpu.CompilerParams(
            dimension_semantics=("parallel",)),
    )(f2, wih, bh, who, bo)


def _mlp2a_body(mx_ref, f_ref, ewih_ref, ebh_ref, ewho_ref, ebo_ref,
                wih_ref, bh_ref, who_ref, bo_ref, o_ref, gm_ref):
    c = mx_ref.shape[2]
    mxv = jnp.max(mx_ref[...], axis=0)
    f = f_ref[...]
    preds_sum = jnp.zeros((f_ref.shape[0], 1), jnp.float32)
    for m in range(4):
        lg = _ens_logits(f, ewih_ref, ebh_ref, ewho_ref, ebo_ref, m)
        e = jnp.exp(lg - jnp.max(mxv[m]))
        p = e / jnp.sum(e, axis=1, keepdims=True)
        pm = jnp.max(p, axis=1, keepdims=True)
        colc = lax.broadcasted_iota(jnp.int32, p.shape, 1)
        am = jnp.min(jnp.where(p == pm, colc, jnp.int32(c)), axis=1,
                     keepdims=True)
        am = jnp.where(jnp.isnan(pm), jnp.int32(0), am)
        preds_sum = preds_sum + am.astype(jnp.float32)
    agg = preds_sum * 0.25
    w = f * agg
    h2 = jnp.maximum(
        jnp.dot(w, wih_ref[...], preferred_element_type=jnp.float32)
        + bh_ref[...][None, :], 0.0)
    o = (jnp.dot(h2, who_ref[...], preferred_element_type=jnp.float32)
         + bo_ref[...][None, :])
    o_ref[...] = o
    gm_ref[...] = jnp.broadcast_to(jnp.max(o), gm_ref.shape)


def _mlp2a(mx, f2, ewih, ebh, ewho, ebo, wih, bh, who, bo):
    c = mx.shape[2]
    nb = _N // _BR2
    return pl.pallas_call(
        _mlp2a_body,
        grid=(nb,),
        in_specs=[
            pl.BlockSpec(mx.shape, lambda i: (0, 0, 0)),
            pl.BlockSpec((_BR2, _D), lambda i: (i, 0)),
            pl.BlockSpec(ewih.shape, lambda i: (0, 0, 0)),
            pl.BlockSpec(ebh.shape, lambda i: (0, 0)),
            pl.BlockSpec(ewho.shape, lambda i: (0, 0, 0)),
            pl.BlockSpec(ebo.shape, lambda i: (0, 0)),
            pl.BlockSpec(wih.shape, lambda i: (0, 0)),
            pl.BlockSpec(bh.shape, lambda i: (0,)),
            pl.BlockSpec(who.shape, lambda i: (0, 0)),
            pl.BlockSpec(bo.shape, lambda i: (0,)),
        ],
        out_specs=[
            pl.BlockSpec((_BR2, c), lambda i: (i, 0)),
            pl.BlockSpec((1, 8, 128), lambda i: (i, 0, 0)),
        ],
        out_shape=[
            jax.ShapeDtypeStruct((_N, c), jnp.float32),
            jax.ShapeDtypeStruct((nb, 8, 128), jnp.float32),
        ],
        compiler_params=pltpu.CompilerParams(
            dimension_semantics=("parallel",)),
    )(mx, f2, ewih, ebh, ewho, ebo, wih, bh, who, bo)


def _mlp2b_body(o_ref, gm_ref, out_ref):
    g = jnp.max(gm_ref[...])
    e2 = jnp.exp(o_ref[...] - g)
    out_ref[...] = e2 / jnp.sum(e2, axis=1, keepdims=True)


def _mlp2b(o, gm):
    c = o.shape[1]
    return pl.pallas_call(
        _mlp2b_body,
        grid=(_N // _BR2,),
        in_specs=[
            pl.BlockSpec((_BR2, c), lambda i: (i, 0)),
            pl.BlockSpec(gm.shape, lambda i: (0, 0, 0)),
        ],
        out_specs=pl.BlockSpec((_BR2, c), lambda i: (i, 0)),
        out_shape=jax.ShapeDtypeStruct((_N, c), jnp.float32),
        compiler_params=pltpu.CompilerParams(
            dimension_semantics=("parallel",)),
    )(o, gm)


def kernel(adjacency_matrix, node_features, ens_W_ih, ens_b_h, ens_W_ho,
           ens_b_o, clf_W_ih, clf_b_h, clf_W_ho, clf_b_o):
    idx8 = _top2(adjacency_matrix)
    i0 = idx8[:, 0]
    i1 = idx8[:, 1]
    f1 = _sc_combine(node_features, i0, i1)
    f2 = _sc_combine(f1, i0, i1)
    mx = _mlp1(f2, ens_W_ih, ens_b_h, ens_W_ho, ens_b_o)
    o, gm = _mlp2a(mx, f2, ens_W_ih, ens_b_h, ens_W_ho, ens_b_o,
                   clf_W_ih, clf_b_h, clf_W_ho, clf_b_o)
    return _mlp2b(o, gm)


# speedup vs baseline: 1.2155x; 1.0684x over previous
import functools

import jax
import jax.numpy as jnp
from jax import lax
from jax.experimental import pallas as pl
from jax.experimental.pallas import tpu as pltpu
from jax.experimental.pallas import tpu_sc as plsc

_N = 10000
_D = 128
_BR = 1000
_BC = 2048
_NCB = 5

_WK = 25
_RPW = 400
_CH = 200

_BR1 = 2000
_BR2 = 2000


def _top2_body(a_ref, out_ref, rv1, ri1, rv2, ri2):
    j = pl.program_id(1)
    nj = pl.num_programs(1)
    v = a_ref[...]
    colf = lax.broadcasted_iota(jnp.int32, v.shape, 1).astype(jnp.float32)
    limit = (_N - j * _BC).astype(jnp.float32)
    v = jnp.where(colf < limit, v, -jnp.inf)
    bigf = jnp.float32(4096.0)
    m1 = jnp.max(v, axis=1, keepdims=True)
    i1f = jnp.min(jnp.where(v == m1, colf, bigf), axis=1, keepdims=True)
    v2 = jnp.where(colf == i1f, -jnp.inf, v)
    m2 = jnp.max(v2, axis=1, keepdims=True)
    i2f = jnp.min(jnp.where(v2 == m2, colf, bigf), axis=1, keepdims=True)
    off = j * _BC
    i1 = i1f.astype(jnp.int32) + off
    i2 = i2f.astype(jnp.int32) + off

    @pl.when(j == 0)
    def _init():
        rv1[...] = m1
        ri1[...] = i1
        rv2[...] = m2
        ri2[...] = i2

    @pl.when(j > 0)
    def _merge():
        pv1, pi1 = rv1[...], ri1[...]
        pv2, pi2 = rv2[...], ri2[...]
        b1 = m1 > pv1
        cav = jnp.where(b1, pv1, pv2)
        cai = jnp.where(b1, pi1, pi2)
        cbv = jnp.where(b1, m2, m1)
        cbi = jnp.where(b1, i2, i1)
        b2 = cbv > cav
        rv1[...] = jnp.where(b1, m1, pv1)
        ri1[...] = jnp.where(b1, i1, pi1)
        rv2[...] = jnp.where(b2, cbv, cav)
        ri2[...] = jnp.where(b2, cbi, cai)

    @pl.when(j == nj - 1)
    def _write():
        lane = lax.broadcasted_iota(jnp.int32, (out_ref.shape[0], 8), 1)
        out_ref[...] = jnp.where(lane == 0, ri1[...],
                                 jnp.where(lane == 1, ri2[...], 0))


def _top2(adjacency):
    return pl.pallas_call(
        _top2_body,
        grid=(_N // _BR, _NCB),
        in_specs=[pl.BlockSpec((_BR, _BC), lambda i, j: (i, j))],
        out_specs=pl.BlockSpec((_BR, 8), lambda i, j: (i, 0)),
        out_shape=jax.ShapeDtypeStruct((_N, 8), jnp.int32),
        scratch_shapes=[
            pltpu.VMEM((_BR, 1), jnp.float32),
            pltpu.VMEM((_BR, 1), jnp.int32),
            pltpu.VMEM((_BR, 1), jnp.float32),
            pltpu.VMEM((_BR, 1), jnp.int32),
        ],
        compiler_params=pltpu.CompilerParams(
            dimension_semantics=("parallel", "arbitrary")),
    )(adjacency)


def _sc_combine_body(table_hbm, i0_hbm, i1_hbm, out_hbm,
                     i0_v, i1_v, self_v, g0_v, g1_v, sem0, sem1, sem2):
    wid = lax.axis_index("s") * 2 + lax.axis_index("c")

    @pl.when(wid < _WK)
    def _work():
        for s in range(_RPW // _CH):
            b = wid * _RPW + s * _CH
            cp0 = pltpu.async_copy(i0_hbm.at[pl.ds(b, _CH)], i0_v, sem0)
            cp1 = pltpu.async_copy(i1_hbm.at[pl.ds(b, _CH)], i1_v, sem1)
            cps = pltpu.async_copy(table_hbm.at[pl.ds(b, _CH)], self_v, sem2)
            cp0.wait()
            cp1.wait()
            g0c = pltpu.async_copy(table_hbm.at[i0_v], g0_v, sem0)
            g1c = pltpu.async_copy(table_hbm.at[i1_v], g1_v, sem1)
            cps.wait()
            g0c.wait()
            g1c.wait()

            def row(r, _):
                for c in range(_D // 16):
                    sl = pl.ds(c * 16, 16)
                    agg = (g0_v[r, sl] + g1_v[r, sl]) * 0.5
                    self_v[r, sl] = (self_v[r, sl] + agg) * 0.5
                return 0

            lax.fori_loop(0, _CH, row, 0)
            pltpu.sync_copy(self_v, out_hbm.at[pl.ds(b, _CH)])


def _sc_combine(table, i0, i1):
    mesh = plsc.VectorSubcoreMesh(core_axis_name="c", subcore_axis_name="s")
    fn = functools.partial(
        pl.kernel,
        out_type=jax.ShapeDtypeStruct((_N, _D), jnp.float32),
        mesh=mesh,
        scratch_types=[
            pltpu.VMEM((_CH,), jnp.int32),
            pltpu.VMEM((_CH,), jnp.int32),
            pltpu.VMEM((_CH, _D), jnp.float32),
            pltpu.VMEM((_CH, _D), jnp.float32),
            pltpu.VMEM((_CH, _D), jnp.float32),
            pltpu.SemaphoreType.DMA,
            pltpu.SemaphoreType.DMA,
            pltpu.SemaphoreType.DMA,
        ],
    )(_sc_combine_body)
    return fn(table, i0, i1)


def _ens_logits(f, wih_ref, bh_ref, who_ref, bo_ref, m):
    hid = jnp.maximum(
        jnp.dot(f, wih_ref[m], preferred_element_type=jnp.float32)
        + bh_ref[m][None, :], 0.0)
    return (jnp.dot(hid, who_ref[m], preferred_element_type=jnp.float32)
            + bo_ref[m][None, :])


def _mlp1_body(f_ref, wih_ref, bh_ref, who_ref, bo_ref, mx_ref):
    f = f_ref[...]
    bms = []
    for m in range(4):
        lg = _ens_logits(f, wih_ref, bh_ref, who_ref, bo_ref, m)
        bms.append(jnp.max(lg, axis=0))
    mx_ref[0] = jnp.stack(bms, axis=0)


def _mlp1(f2, wih, bh, who, bo):
    c = who.shape[2]
    nb = _N // _BR1
    return pl.pallas_call(
        _mlp1_body,
        grid=(nb,),
        in_specs=[
            pl.BlockSpec((_BR1, _D), lambda i: (i, 0)),
            pl.BlockSpec(wih.shape, lambda i: (0, 0, 0)),
            pl.BlockSpec(bh.shape, lambda i: (0, 0)),
            pl.BlockSpec(who.shape, lambda i: (0, 0, 0)),
            pl.BlockSpec(bo.shape, lambda i: (0, 0)),
        ],
        out_specs=pl.BlockSpec((1, 4, c), lambda i: (i, 0, 0)),
        out_shape=jax.ShapeDtypeStruct((nb, 4, c), jnp.float32),
        compiler_params=pltpu.CompilerParams(
            dimension_semantics=("parallel",)),
    )(f2, wih, bh, who, bo)


def _mlp2a_body(mx_ref, f_ref, ewih_ref, ebh_ref, ewho_ref, ebo_ref,
                wih_ref, bh_ref, who_ref, bo_ref, o_ref, gm_ref):
    c = mx_ref.shape[2]
    mxv = jnp.max(mx_ref[...], axis=0)
    f = f_ref[...]
    preds_sum = jnp.zeros((f_ref.shape[0], 1), jnp.float32)
    for m in range(4):
        lg = _ens_logits(f, ewih_ref, ebh_ref, ewho_ref, ebo_ref, m)
        e = jnp.exp(lg - jnp.max(mxv[m]))
        p = e / jnp.sum(e, axis=1, keepdims=True)
        pm = jnp.max(p, axis=1, keepdims=True)
        colc = lax.broadcasted_iota(jnp.int32, p.shape, 1)
        am = jnp.min(jnp.where(p == pm, colc, jnp.int32(c)), axis=1,
                     keepdims=True)
        am = jnp.where(jnp.isnan(pm), jnp.int32(0), am)
        preds_sum = preds_sum + am.astype(jnp.float32)
    agg = preds_sum * 0.25
    w = f * agg
    h2 = jnp.maximum(
        jnp.dot(w, wih_ref[...], preferred_element_type=jnp.float32)
        + bh_ref[...][None, :], 0.0)
    o = (jnp.dot(h2, who_ref[...], preferred_element_type=jnp.float32)
         + bo_ref[...][None, :])
    o_ref[...] = o
    gm_ref[...] = jnp.broadcast_to(jnp.max(o), gm_ref.shape)


def _mlp2a(mx, f2, ewih, ebh, ewho, ebo, wih, bh, who, bo):
    c = mx.shape[2]
    nb = _N // _BR2
    return pl.pallas_call(
        _mlp2a_body,
        grid=(nb,),
        in_specs=[
            pl.BlockSpec(mx.shape, lambda i: (0, 0, 0)),
            pl.BlockSpec((_BR2, _D), lambda i: (i, 0)),
            pl.BlockSpec(ewih.shape, lambda i: (0, 0, 0)),
            pl.BlockSpec(ebh.shape, lambda i: (0, 0)),
            pl.BlockSpec(ewho.shape, lambda i: (0, 0, 0)),
            pl.BlockSpec(ebo.shape, lambda i: (0, 0)),
            pl.BlockSpec(wih.shape, lambda i: (0, 0)),
            pl.BlockSpec(bh.shape, lambda i: (0,)),
            pl.BlockSpec(who.shape, lambda i: (0, 0)),
            pl.BlockSpec(bo.shape, lambda i: (0,)),
        ],
        out_specs=[
            pl.BlockSpec((_BR2, c), lambda i: (i, 0)),
            pl.BlockSpec((1, 8, 128), lambda i: (i, 0, 0)),
        ],
        out_shape=[
            jax.ShapeDtypeStruct((_N, c), jnp.float32),
            jax.ShapeDtypeStruct((nb, 8, 128), jnp.float32),
        ],
        compiler_params=pltpu.CompilerParams(
            dimension_semantics=("parallel",)),
    )(mx, f2, ewih, ebh, ewho, ebo, wih, bh, who, bo)


def _mlp2b_body(o_ref, gm_ref, out_ref):
    g = jnp.max(gm_ref[...])
    e2 = jnp.exp(o_ref[...] - g)
    out_ref[...] = e2 / jnp.sum(e2, axis=1, keepdims=True)


def _mlp2b(o, gm):
    c = o.shape[1]
    return pl.pallas_call(
        _mlp2b_body,
        grid=(_N // _BR2,),
        in_specs=[
            pl.BlockSpec((_BR2, c), lambda i: (i, 0)),
            pl.BlockSpec(gm.shape, lambda i: (0, 0, 0)),
        ],
        out_specs=pl.BlockSpec((_BR2, c), lambda i: (i, 0)),
        out_shape=jax.ShapeDtypeStruct((_N, c), jnp.float32),
        compiler_params=pltpu.CompilerParams(
            dimension_semantics=("parallel",)),
    )(o, gm)


def kernel(adjacency_matrix, node_features, ens_W_ih, ens_b_h, ens_W_ho,
           ens_b_o, clf_W_ih, clf_b_h, clf_W_ho, clf_b_o):
    idx8 = _top2(adjacency_matrix)
    i0 = idx8[:, 0]
    i1 = idx8[:, 1]
    f1 = _sc_combine(node_features, i0, i1)
    f2 = _sc_combine(f1, i0, i1)
    mx = _mlp1(f2, ens_W_ih, ens_b_h, ens_W_ho, ens_b_o)
    o, gm = _mlp2a(mx, f2, ens_W_ih, ens_b_h, ens_W_ho, ens_b_o,
                   clf_W_ih, clf_b_h, clf_W_ho, clf_b_o)
    return _mlp2b(o, gm)
